# baseline (speedup 1.0000x reference)
import jax
import jax.numpy as jnp
from jax import lax
from jax.experimental import pallas as pl
from jax.experimental.pallas import tpu as pltpu
from jax.experimental.pallas import tpu_sc as plsc

TOP_K = 2
_NEG = -3.0e38

B, P, S, D = 4, 8, 2048, 1024
NW = 32
WPB = NW // B
CHUNK = S // WPB
R = 4
RE = R * D
NST = CHUNK // R
PLANE = S * D


def _sc_stream_body(probes_hbm, scores_hbm, bias_hbm, out_hbm,
                    scores_v, bias_v, p0v, p1v, mb, ob,
                    sem_in0, sem_in1, sem_out):
    cid = lax.axis_index("c")
    sid = lax.axis_index("s")
    wid = cid * 16 + sid
    b = wid // WPB
    chunk = wid % WPB
    base = (chunk * CHUNK) * D

    pltpu.sync_copy(scores_hbm, scores_v)
    pltpu.sync_copy(bias_hbm, bias_v)
    lanes = lax.iota(jnp.int32, 16)
    half = b // 2
    sub = b % 2
    pair = scores_v[pl.ds(half * 16, 16)]
    in_row = (lanes >= 8 * sub) & (lanes < 8 * sub + 8)
    v = jnp.where(in_row, pair, _NEG)
    sk, sv = plsc.sort_key_val(v, lanes - 8 * sub, descending=True)
    m1 = sk[0]
    m2 = sk[1]
    i0 = sv[0]
    i1 = sv[1]
    ev = jnp.exp(jnp.full((16,), m2 - m1, jnp.float32))
    w0v = 1.0 / (1.0 + ev)
    w0 = w0v[0]
    w1 = 1.0 - w0

    def in_copies(st, par):
        off = base + st * RE
        return (
            pltpu.make_async_copy(
                probes_hbm.at[b, i0, pl.ds(off, RE)], p0v.at[par], sem_in0),
            pltpu.make_async_copy(
                probes_hbm.at[b, i1, pl.ds(off, RE)], p1v.at[par], sem_in1),
        )

    def out_copy(st, par, p):
        off = base + st * RE
        return pltpu.make_async_copy(
            ob.at[par, p], out_hbm.at[b, p, pl.ds(off, RE)], sem_out.at[par])

    def start_in(st, par):
        for c in in_copies(st, par):
            c.start()

    def do_subtile(st, par):
        @pl.when(st + 1 < NST)
        def _():
            start_in(st + 1, 1 - par)
        for c in in_copies(st, par):
            c.wait()

        @plsc.parallel_loop(0, RE // 16, unroll=4)
        def _(i):
            o = i * 16
            mb[pl.ds(o, 16)] = (p0v[par, pl.ds(o, 16)] * w0
                                + p1v[par, pl.ds(o, 16)] * w1)

        @pl.when(st >= 2)
        def _():
            for p in range(P):
                out_copy(st - 2, par, p).wait()

        @plsc.parallel_loop(0, P * (RE // 16), unroll=4)
        def _(i):
            p = i >> 8
            o = (i & 255) * 16
            ob[par, p, pl.ds(o, 16)] = (
                mb[pl.ds(o, 16)] + bias_v[pl.ds((p << 10) + (o & 1023), 16)])

        for p in range(P):
            out_copy(st, par, p).start()

    start_in(0, 0)

    def body(k, carry):
        do_subtile(2 * k, 0)
        do_subtile(2 * k + 1, 1)
        return carry

    lax.fori_loop(0, NST // 2, body, jnp.int32(0))

    for p in range(P):
        out_copy(NST - 2, 0, p).wait()
    for p in range(P):
        out_copy(NST - 1, 1, p).wait()


def kernel(probes, scores, re_expand):
    Bb, Pp, Ss, Dd = probes.shape
    mesh = plsc.VectorSubcoreMesh(core_axis_name="c", subcore_axis_name="s")
    out_flat = pl.kernel(
        _sc_stream_body,
        out_type=jax.ShapeDtypeStruct((Bb, Pp, Ss * Dd), jnp.float32),
        mesh=mesh,
        compiler_params=pltpu.CompilerParams(needs_layout_passes=False),
        scratch_types=[
            pltpu.VMEM((32,), jnp.float32),
            pltpu.VMEM((P * D,), jnp.float32),
            pltpu.VMEM((2, RE), jnp.float32),
            pltpu.VMEM((2, RE), jnp.float32),
            pltpu.VMEM((RE,), jnp.float32),
            pltpu.VMEM((2, P, RE), jnp.float32),
            pltpu.SemaphoreType.DMA,
            pltpu.SemaphoreType.DMA,
            pltpu.SemaphoreType.DMA((2,)),
        ],
    )(probes.reshape(Bb, Pp, Ss * Dd), scores.reshape(-1), re_expand.reshape(-1))
    return out_flat.reshape(Bb, Pp, Ss, Dd)


# speedup vs baseline: 53.5863x; 53.5863x over previous
import jax
import jax.numpy as jnp
from jax import lax
from jax.experimental import pallas as pl
from jax.experimental.pallas import tpu as pltpu
from jax.experimental.pallas import tpu_sc as plsc

TOP_K = 2
TS = 512
_NEG = -3.0e38


def _sc_routing_body(scores_hbm, idx_hbm, w_hbm, scores_v, idx_v, w_v):
    cid = lax.axis_index("c")
    sid = lax.axis_index("s")

    @pl.when(jnp.logical_and(cid == 0, sid == 0))
    def _():
        pltpu.sync_copy(scores_hbm, scores_v)
        lanes = lax.iota(jnp.int32, 16)
        idx_acc = jnp.zeros((16,), jnp.int32)
        w_acc = jnp.zeros((16,), jnp.float32)
        for half in range(2):
            pair = scores_v[pl.ds(half * 16, 16)]
            for sub in range(2):
                b = 2 * half + sub
                in_row = (lanes >= 8 * sub) & (lanes < 8 * (sub + 1))
                v = jnp.where(in_row, pair, _NEG)
                sk, sv = plsc.sort_key_val(v, lanes - 8 * sub, descending=True)
                m1 = sk[0]
                m2 = sk[1]
                i0 = sv[0]
                i1 = sv[1]
                ev = jnp.exp(jnp.full((16,), m2 - m1, jnp.float32))
                w0v = 1.0 / (1.0 + ev)
                w0 = w0v[0]
                w1 = 1.0 - w0
                idx_acc = jnp.where(lanes == 2 * b, i0, idx_acc)
                idx_acc = jnp.where(lanes == 2 * b + 1, i1, idx_acc)
                w_acc = jnp.where(lanes == 2 * b, w0, w_acc)
                w_acc = jnp.where(lanes == 2 * b + 1, w1, w_acc)
        idx_v[...] = idx_acc
        w_v[...] = w_acc
        pltpu.sync_copy(idx_v, idx_hbm)
        pltpu.sync_copy(w_v, w_hbm)


def _sc_routing(scores_flat):
    mesh = plsc.VectorSubcoreMesh(
        core_axis_name="c", subcore_axis_name="s", num_cores=1, num_subcores=1
    )
    return pl.kernel(
        _sc_routing_body,
        out_type=(
            jax.ShapeDtypeStruct((16,), jnp.int32),
            jax.ShapeDtypeStruct((16,), jnp.float32),
        ),
        mesh=mesh,
        compiler_params=pltpu.CompilerParams(needs_layout_passes=False),
        scratch_types=[
            pltpu.VMEM((32,), jnp.float32),
            pltpu.VMEM((16,), jnp.int32),
            pltpu.VMEM((16,), jnp.float32),
        ],
    )(scores_flat)


def _fold_kernel(idx_ref, w_ref, p0_ref, p1_ref, reexp_ref, out_ref):
    b = pl.program_id(0)
    w0 = w_ref[2 * b]
    w1 = w_ref[2 * b + 1]
    merged = p0_ref[0, 0] * w0 + p1_ref[0, 0] * w1
    for p in range(out_ref.shape[1]):
        out_ref[0, p] = merged + reexp_ref[p]


def kernel(probes, scores, re_expand):
    B, P, S, D = probes.shape
    idx16, w16 = _sc_routing(scores.reshape(-1))

    grid = (B, S // TS)

    def probe_spec(k):
        def imap(b, s, idx_ref, w_ref):
            return (b, idx_ref[2 * b + k], s, 0)
        return pl.BlockSpec((1, 1, TS, D), imap)

    out_spec = pl.BlockSpec((1, P, TS, D), lambda b, s, idx_ref, w_ref: (b, 0, s, 0))
    reexp_spec = pl.BlockSpec((P, D), lambda b, s, idx_ref, w_ref: (0, 0))

    grid_spec = pltpu.PrefetchScalarGridSpec(
        num_scalar_prefetch=2,
        grid=grid,
        in_specs=[probe_spec(0), probe_spec(1), reexp_spec],
        out_specs=out_spec,
    )

    return pl.pallas_call(
        _fold_kernel,
        grid_spec=grid_spec,
        out_shape=jax.ShapeDtypeStruct((B, P, S, D), probes.dtype),
        compiler_params=pltpu.CompilerParams(
            dimension_semantics=("parallel", "arbitrary"),
        ),
    )(idx16, w16, probes, probes, re_expand)
